# baseline (speedup 1.0000x reference)
import jax
import jax.numpy as jnp
from jax.experimental import pallas as pl
from jax.experimental.pallas import tpu as pltpu


def _body(row_ref, col_ref, out_ref, slab_ref):
    h = row_ref.shape[0]
    m = h * h

    @pl.when(pl.program_id(0) == 0)
    def _():
        m_ids = jax.lax.broadcasted_iota(jnp.int32, (h, m), 1)
        r_ids = jax.lax.broadcasted_iota(jnp.int32, (h, m), 0)
        sel_q = (m_ids % h == r_ids).astype(jnp.float32)
        sel_p = (m_ids // h == r_ids).astype(jnp.float32)
        dn = (((0,), (0,)), ((), ()))
        top = jax.lax.dot_general(row_ref[...], sel_q, dn,
                                  precision=jax.lax.Precision.HIGHEST)
        bot = jax.lax.dot_general(col_ref[...], sel_p, dn,
                                  precision=jax.lax.Precision.HIGHEST)
        slab = jnp.concatenate([top, bot], axis=0)
        slab_ref[...] = slab.reshape(slab_ref.shape)

    out_ref[...] = jnp.broadcast_to(slab_ref[...][None], out_ref.shape)


def kernel(x, row_table, col_table):
    bs, _, h, w = x.shape
    out_n = row_table.shape[1]
    c_total = 2 * out_n
    rows = c_total * h * w // 128
    bblk = 2

    flat = pl.pallas_call(
        _body,
        grid=(bs // bblk,),
        in_specs=[
            pl.BlockSpec((h, out_n), lambda b: (0, 0)),
            pl.BlockSpec((w, out_n), lambda b: (0, 0)),
        ],
        out_specs=pl.BlockSpec((bblk, rows, 128), lambda b: (b, 0, 0)),
        out_shape=jax.ShapeDtypeStruct((bs, rows, 128), jnp.float32),
        scratch_shapes=[pltpu.VMEM((rows, 128), jnp.float32)],
    )(row_table[:h], col_table[:w])
    return flat.reshape(bs, c_total, h, w)


# speedup vs baseline: 13.5961x; 13.5961x over previous
import jax
import jax.numpy as jnp
from jax.experimental import pallas as pl


def _body(row_ref, col_ref, out_ref):
    h, out_n = row_ref.shape
    top = jnp.broadcast_to(row_ref[...][None, :, :], (h, h, out_n))
    bot = jnp.broadcast_to(col_ref[...][:, None, :], (h, h, out_n))
    slab = jnp.concatenate([top, bot], axis=2)
    out_ref[...] = jnp.broadcast_to(slab[None], out_ref.shape)


def kernel(x, row_table, col_table):
    bs, _, h, w = x.shape
    out_n = row_table.shape[1]
    c_total = 2 * out_n
    bblk = 2

    y = pl.pallas_call(
        _body,
        grid=(bs // bblk,),
        in_specs=[
            pl.BlockSpec((h, out_n), lambda b: (0, 0)),
            pl.BlockSpec((w, out_n), lambda b: (0, 0)),
        ],
        out_specs=pl.BlockSpec((bblk, h, w, c_total), lambda b: (b, 0, 0, 0)),
        out_shape=jax.ShapeDtypeStruct((bs, h, w, c_total), jnp.float32),
    )(row_table, col_table)
    return jnp.transpose(y, (0, 3, 1, 2))
